# baseline (speedup 1.0000x reference)
import jax
import jax.numpy as jnp
from jax.experimental import pallas as pl
from jax.experimental.pallas import tpu as pltpu

_ROWS = 32 * 3 * 512
_COLS = 512
_BLOCK_ROWS = 2048
_GRID = _ROWS // _BLOCK_ROWS


def _masked_mse_block(inp_ref, tgt_ref, msk_ref, sum_ref, cnt_ref):
    d = inp_ref[...] - tgt_ref[...]
    m = msk_ref[...] == 1
    sum_ref[...] = jnp.sum(jnp.where(m, d * d, 0.0), axis=0, keepdims=True)[None]
    cnt_ref[...] = jnp.sum(m.astype(jnp.int32), axis=0, keepdims=True)[None]


def kernel(input, target, mask):
    x = input.reshape(_ROWS, _COLS)
    t = target.reshape(_ROWS, _COLS)
    mk = mask.reshape(_ROWS, _COLS)

    in_spec = pl.BlockSpec((_BLOCK_ROWS, _COLS), lambda i: (i, 0))
    out_spec = pl.BlockSpec((1, 1, _COLS), lambda i: (i, 0, 0))

    sums, cnts = pl.pallas_call(
        _masked_mse_block,
        grid=(_GRID,),
        in_specs=[in_spec, in_spec, in_spec],
        out_specs=[out_spec, out_spec],
        out_shape=[
            jax.ShapeDtypeStruct((_GRID, 1, _COLS), jnp.float32),
            jax.ShapeDtypeStruct((_GRID, 1, _COLS), jnp.int32),
        ],
        compiler_params=pltpu.CompilerParams(
            dimension_semantics=("parallel",),
        ),
        name="masked_mse",
    )(x, t, mk)

    return jnp.sum(sums) / jnp.sum(cnts).astype(jnp.float32)
